# baseline (device time: 81566 ns/iter reference)
import jax
import jax.numpy as jnp
from jax import lax
from jax.experimental import pallas as pl
from jax.experimental.pallas import tpu as pltpu

N_DEV = 4
SQ = 2048
SKV = 2048
HQ = 8
DH = 128
DM = HQ * DH
BLK = 64
NRES = 4
NJ = 8
GRP = NJ * BLK
HGRP = GRP // 2
HALF = SQ // 2
QTR = SQ // 4
PW = DM + DH
SCALE = 0.08838834764831843


def _g(qb):
    return (qb % NRES) * GRP + (qb // NRES) * BLK


def kernel(x, Wq, K_ext, V_ext, Wo):
    x2 = x.reshape(SQ, DM)
    K2 = K_ext.reshape(SKV, HQ, DH)
    V2 = V_ext.reshape(SKV, HQ, DH)

    def body(x_ref, wq_ref, k_ref, v_ref, wo_ref, out_ref,
             stage_a, stage_b, stage3_a, stage3_b, wqb, wob,
             xg, qg, kg, vg, acc,
             sbr, rbr, sbl, rbl, agr, agl,
             dma_sems, send_sems, recv_sems):
        my = lax.axis_index("i")
        left = lax.rem(my + N_DEV - 1, N_DEV)
        right = lax.rem(my + 1, N_DEV)

        def load_half(src, half, stage, sem):
            cp = pltpu.make_async_copy(
                src.at[pl.ds(half * HALF, HALF), :], stage, sem)
            cp.start()
            return cp

        def load_full(src, stage, sem):
            cp = pltpu.make_async_copy(src, stage, sem)
            cp.start()
            return cp

        def load_qtr(src, qtr, stage, sem):
            cp = pltpu.make_async_copy(
                src.at[pl.ds(qtr * QTR, QTR), :, :], stage, sem)
            cp.start()
            return cp

        def cast_group(stage, dst, half):
            for bb in range(16):
                b = half * 16 + bb
                dst[_g(b):_g(b) + BLK, :] = \
                    stage[bb * BLK:(bb + 1) * BLK, :].astype(jnp.bfloat16)

        def cast_group3(stage, dst, qtr):
            for bb in range(8):
                b = qtr * 8 + bb
                blk = stage[bb * BLK:(bb + 1) * BLK, :, :]
                dst[_g(b):_g(b) + BLK, :] = \
                    jnp.reshape(blk, (BLK, DM)).astype(jnp.bfloat16)

        sem_a, sem_b = dma_sems.at[0], dma_sems.at[1]
        sem_c, sem_d = dma_sems.at[2], dma_sems.at[3]
        cp_x0 = load_half(x_ref, 0, stage_a, sem_a)
        cp_x1 = load_half(x_ref, 1, stage_b, sem_b)
        cp_k0 = load_qtr(k_ref, 0, stage3_a, sem_c)
        cp_k1 = load_qtr(k_ref, 1, stage3_b, sem_d)

        barrier = pltpu.get_barrier_semaphore()
        for nbr in (left, right):
            pl.semaphore_signal(barrier, inc=1, device_id=(nbr,),
                                device_id_type=pl.DeviceIdType.MESH)
        pl.semaphore_wait(barrier, 2)
        cp_x0.wait()
        cast_group(stage_a, xg, 0)
        cp_wq = load_full(wq_ref, stage_a, sem_a)
        cp_x1.wait()
        cast_group(stage_b, xg, 1)
        cp_wo = load_full(wo_ref, stage_b, sem_b)
        cp_k0.wait()
        cast_group3(stage3_a, kg, 0)
        cp_k2 = load_qtr(k_ref, 2, stage3_a, sem_c)
        cp_k1.wait()
        cast_group3(stage3_b, kg, 1)
        cp_k3 = load_qtr(k_ref, 3, stage3_b, sem_d)
        cp_wq.wait()
        wqb[:, :] = (stage_a[:, :] * SCALE).astype(jnp.bfloat16)

        qg[:, :] = lax.dot_general(
            xg[:, :], wqb[:, :], (((1,), (0,)), ((), ())),
            preferred_element_type=jnp.float32).astype(jnp.bfloat16)

        cp_k2.wait()
        cast_group3(stage3_a, kg, 2)
        cp_v0 = load_qtr(v_ref, 0, stage3_a, sem_c)
        cp_k3.wait()
        cast_group3(stage3_b, kg, 3)
        cp_v1 = load_qtr(v_ref, 1, stage3_b, sem_d)
        cp_v0.wait()
        cast_group3(stage3_a, vg, 0)
        cp_v2 = load_qtr(v_ref, 2, stage3_a, sem_c)
        cp_v1.wait()
        cast_group3(stage3_b, vg, 1)
        cp_v3 = load_qtr(v_ref, 3, stage3_b, sem_d)
        cp_wo.wait()
        wob[:, :] = stage_b[:, :].astype(jnp.bfloat16)
        cp_v2.wait()
        cast_group3(stage3_a, vg, 2)
        cp_v3.wait()
        cast_group3(stage3_b, vg, 3)

        def attn_part(c, qoff, nq):
            qrows = pl.ds(c * GRP + qoff, nq)
            kvrows = pl.ds(c * GRP, GRP)
            for h in range(HQ):
                cols = slice(h * DH, (h + 1) * DH)
                q = qg[qrows, cols]
                k = kg[kvrows, cols]
                s = lax.dot_general(q, k, (((1,), (1,)), ((), ())),
                                    preferred_element_type=jnp.float32)
                w = jnp.exp(s.astype(jnp.bfloat16))
                acc[qrows, DM + h] = jnp.sum(w, axis=1, dtype=jnp.float32)
                acc[qrows, cols] = lax.dot_general(
                    w, vg[kvrows, cols],
                    (((1,), (0,)), ((), ())),
                    preferred_element_type=jnp.float32)

        def attn_chunk(c):
            attn_part(c, 0, GRP)

        attn_part(my, 0, HGRP)
        attn_part(lax.rem(my + 2, N_DEV), HGRP, HGRP)

        def rs_pair(slot):
            r_ = pltpu.make_async_remote_copy(
                src_ref=sbr.at[slot], dst_ref=rbr.at[slot],
                send_sem=send_sems.at[slot], recv_sem=recv_sems.at[slot],
                device_id=(right,), device_id_type=pl.DeviceIdType.MESH)
            l_ = pltpu.make_async_remote_copy(
                src_ref=sbl.at[slot], dst_ref=rbl.at[slot],
                send_sem=send_sems.at[2 + slot], recv_sem=recv_sems.at[2 + slot],
                device_id=(left,), device_id_type=pl.DeviceIdType.MESH)
            return r_, l_

        sbr[0, :, :] = acc[pl.ds(my * GRP, HGRP), :].astype(jnp.bfloat16)
        sbl[0, :, :] = acc[pl.ds(lax.rem(my + 2, N_DEV) * GRP + HGRP, HGRP),
                           :].astype(jnp.bfloat16)
        rdma_r0, rdma_l0 = rs_pair(0)
        rdma_r0.start()
        rdma_l0.start()
        attn_part(my, HGRP, HGRP)
        attn_part(lax.rem(my + 2, N_DEV), 0, HGRP)
        attn_chunk(lax.rem(my + 3, N_DEV))
        rdma_r0.wait()
        rdma_l0.wait()

        c1 = lax.rem(my + 3, N_DEV)
        sbr[1, :, :] = (rbr[0, :, :].astype(jnp.float32)
                        + acc[pl.ds(c1 * GRP, HGRP), :]).astype(jnp.bfloat16)
        sbl[1, :, :] = (rbl[0, :, :].astype(jnp.float32)
                        + acc[pl.ds(c1 * GRP + HGRP, HGRP), :]
                        ).astype(jnp.bfloat16)
        rdma_r1, rdma_l1 = rs_pair(1)
        rdma_r1.start()
        rdma_l1.start()
        attn_chunk(lax.rem(my + 1, N_DEV))
        rdma_r1.wait()
        rdma_l1.wait()

        sbr[0, :, :] = (rbr[1, :, :].astype(jnp.float32)
                        + acc[pl.ds(lax.rem(my + 2, N_DEV) * GRP, HGRP), :]
                        ).astype(jnp.bfloat16)
        sbl[0, :, :] = (rbl[1, :, :].astype(jnp.float32)
                        + acc[pl.ds(my * GRP + HGRP, HGRP), :]
                        ).astype(jnp.bfloat16)
        rdma_r2, rdma_l2 = rs_pair(0)
        rdma_r2.start()
        rdma_l2.start()
        rdma_r2.wait()
        rdma_l2.wait()

        own = lax.rem(my + 1, N_DEV)

        def norm_proj(rb, row_off):
            s_half = (rb[0, :, :].astype(jnp.float32)
                      + acc[pl.ds(own * GRP + row_off, HGRP), :])
            ctx = jnp.concatenate(
                [(s_half[:, h * DH:(h + 1) * DH]
                  * (1.0 / s_half[:, DM + h])[:, None]).astype(jnp.bfloat16)
                 for h in range(HQ)], axis=1)
            return lax.dot_general(
                ctx, wob[:, :], (((1,), (0,)), ((), ())),
                preferred_element_type=jnp.float32)

        def ag_pair(ss, rr):
            r_ = pltpu.make_async_remote_copy(
                src_ref=agr.at[ss], dst_ref=agr.at[rr],
                send_sem=send_sems.at[4 + ss], recv_sem=recv_sems.at[4 + rr],
                device_id=(right,), device_id_type=pl.DeviceIdType.MESH)
            l_ = pltpu.make_async_remote_copy(
                src_ref=agl.at[ss], dst_ref=agl.at[rr],
                send_sem=send_sems.at[6 + ss], recv_sem=recv_sems.at[6 + rr],
                device_id=(left,), device_id_type=pl.DeviceIdType.MESH)
            return r_, l_

        def ag_store(slot, t):
            rho_r = lax.rem(my - t + 2 * N_DEV, N_DEV)
            rho_l = lax.rem(my + t + 2, N_DEV)
            for j in range(NJ // 2):
                out_ref[0, pl.ds(j * NRES * BLK + rho_r * BLK, BLK), :] = \
                    agr[slot, pl.ds(j * BLK, BLK), :].astype(jnp.float32)
                out_ref[0, pl.ds((j + 4) * NRES * BLK + rho_l * BLK, BLK), :] = \
                    agl[slot, pl.ds(j * BLK, BLK), :].astype(jnp.float32)

        out_top = norm_proj(rbr, 0)
        agr[0, :, :] = out_top.astype(jnp.bfloat16)
        ag_r0, ag_l0 = ag_pair(0, 1)
        ag_r0.start()
        out_bot = norm_proj(rbl, HGRP)
        agl[0, :, :] = out_bot.astype(jnp.bfloat16)
        ag_l0.start()
        for j in range(NJ // 2):
            out_ref[0, pl.ds(j * NRES * BLK + own * BLK, BLK), :] = \
                out_top[j * BLK:(j + 1) * BLK, :]
            out_ref[0, pl.ds((j + 4) * NRES * BLK + own * BLK, BLK), :] = \
                out_bot[j * BLK:(j + 1) * BLK, :]
        ag_r0.wait()
        ag_l0.wait()
        ag_r1, ag_l1 = ag_pair(1, 0)
        ag_r1.start()
        ag_l1.start()
        ag_store(1, 0)
        ag_r1.wait()
        ag_l1.wait()
        ag_r2, ag_l2 = ag_pair(0, 1)
        ag_r2.start()
        ag_l2.start()
        ag_store(0, 1)
        ag_r2.wait()
        ag_l2.wait()
        ag_store(1, 2)

    out = pl.pallas_call(
        body,
        out_shape=jax.ShapeDtypeStruct((1, SQ, DM), jnp.float32),
        in_specs=[pl.BlockSpec(memory_space=pltpu.MemorySpace.HBM)] * 5,
        out_specs=pl.BlockSpec(memory_space=pltpu.VMEM),
        scratch_shapes=[
            pltpu.VMEM((HALF, DM), jnp.float32),
            pltpu.VMEM((HALF, DM), jnp.float32),
            pltpu.VMEM((QTR, HQ, DH), jnp.float32),
            pltpu.VMEM((QTR, HQ, DH), jnp.float32),
            pltpu.VMEM((DM, DM), jnp.bfloat16),
            pltpu.VMEM((DM, DM), jnp.bfloat16),
            pltpu.VMEM((SQ, DM), jnp.bfloat16),
            pltpu.VMEM((SQ, DM), jnp.bfloat16),
            pltpu.VMEM((SKV, DM), jnp.bfloat16),
            pltpu.VMEM((SKV, DM), jnp.bfloat16),
            pltpu.VMEM((SQ, PW), jnp.float32),
            pltpu.VMEM((2, HGRP, PW), jnp.bfloat16),
            pltpu.VMEM((2, HGRP, PW), jnp.bfloat16),
            pltpu.VMEM((2, HGRP, PW), jnp.bfloat16),
            pltpu.VMEM((2, HGRP, PW), jnp.bfloat16),
            pltpu.VMEM((2, HGRP, DM), jnp.bfloat16),
            pltpu.VMEM((2, HGRP, DM), jnp.bfloat16),
            pltpu.SemaphoreType.DMA((4,)),
            pltpu.SemaphoreType.DMA((8,)),
            pltpu.SemaphoreType.DMA((8,)),
        ],
        compiler_params=pltpu.CompilerParams(
            collective_id=0, vmem_limit_bytes=60 * 1024 * 1024),
    )(x2, Wq, K2, V2, Wo)

    return out


# device time: 81501 ns/iter; 1.0008x vs baseline; 1.0008x over previous
import jax
import jax.numpy as jnp
from jax import lax
from jax.experimental import pallas as pl
from jax.experimental.pallas import tpu as pltpu

N_DEV = 4
SQ = 2048
SKV = 2048
HQ = 8
DH = 128
DM = HQ * DH
BLK = 64
NRES = 4
NJ = 8
GRP = NJ * BLK
HGRP = GRP // 2
HALF = SQ // 2
QTR = SQ // 4
PW = DM + DH
SCALE = 0.08838834764831843


def _g(qb):
    return (qb % NRES) * GRP + (qb // NRES) * BLK


def kernel(x, Wq, K_ext, V_ext, Wo):
    x2 = x.reshape(SQ, DM)
    K2 = K_ext.reshape(SKV, HQ, DH)
    V2 = V_ext.reshape(SKV, HQ, DH)

    def body(x_ref, wq_ref, k_ref, v_ref, wo_ref, out_ref,
             stage_a, stage_b, stage3_a, stage3_b, wqb, wob,
             xg, qg, kg, vg, acc,
             sbr, rbr, sbl, rbl, agr, agl,
             dma_sems, send_sems, recv_sems):
        my = lax.axis_index("i")
        left = lax.rem(my + N_DEV - 1, N_DEV)
        right = lax.rem(my + 1, N_DEV)

        def load_half(src, half, stage, sem):
            cp = pltpu.make_async_copy(
                src.at[pl.ds(half * HALF, HALF), :], stage, sem)
            cp.start()
            return cp

        def load_full(src, stage, sem):
            cp = pltpu.make_async_copy(src, stage, sem)
            cp.start()
            return cp

        def load_qtr(src, qtr, stage, sem):
            cp = pltpu.make_async_copy(
                src.at[pl.ds(qtr * QTR, QTR), :, :], stage, sem)
            cp.start()
            return cp

        def cast_group(stage, dst, half):
            for bb in range(16):
                b = half * 16 + bb
                dst[_g(b):_g(b) + BLK, :] = \
                    stage[bb * BLK:(bb + 1) * BLK, :].astype(jnp.bfloat16)

        def cast_group3(stage, dst, qtr):
            for bb in range(8):
                b = qtr * 8 + bb
                blk = stage[bb * BLK:(bb + 1) * BLK, :, :]
                dst[_g(b):_g(b) + BLK, :] = \
                    jnp.reshape(blk, (BLK, DM)).astype(jnp.bfloat16)

        sem_a, sem_b = dma_sems.at[0], dma_sems.at[1]
        sem_c, sem_d = dma_sems.at[2], dma_sems.at[3]
        cp_x0 = load_half(x_ref, 0, stage_a, sem_a)
        cp_x1 = load_half(x_ref, 1, stage_b, sem_b)
        cp_k0 = load_qtr(k_ref, 0, stage3_a, sem_c)
        cp_k1 = load_qtr(k_ref, 1, stage3_b, sem_d)

        barrier = pltpu.get_barrier_semaphore()
        for nbr in (left, right):
            pl.semaphore_signal(barrier, inc=1, device_id=(nbr,),
                                device_id_type=pl.DeviceIdType.MESH)
        pl.semaphore_wait(barrier, 2)
        cp_x0.wait()
        cast_group(stage_a, xg, 0)
        cp_wq = load_full(wq_ref, stage_a, sem_a)
        cp_x1.wait()
        cast_group(stage_b, xg, 1)
        cp_wo = load_full(wo_ref, stage_b, sem_b)
        cp_k0.wait()
        cast_group3(stage3_a, kg, 0)
        cp_k2 = load_qtr(k_ref, 2, stage3_a, sem_c)
        cp_k1.wait()
        cast_group3(stage3_b, kg, 1)
        cp_k3 = load_qtr(k_ref, 3, stage3_b, sem_d)
        cp_wq.wait()
        wqb[:, :] = (stage_a[:, :] * SCALE).astype(jnp.bfloat16)

        qg[:, :] = lax.dot_general(
            xg[:, :], wqb[:, :], (((1,), (0,)), ((), ())),
            preferred_element_type=jnp.float32).astype(jnp.bfloat16)

        cp_k2.wait()
        cast_group3(stage3_a, kg, 2)
        cp_v0 = load_qtr(v_ref, 0, stage3_a, sem_c)
        cp_k3.wait()
        cast_group3(stage3_b, kg, 3)
        cp_v1 = load_qtr(v_ref, 1, stage3_b, sem_d)
        cp_v0.wait()
        cast_group3(stage3_a, vg, 0)
        cp_v2 = load_qtr(v_ref, 2, stage3_a, sem_c)
        cp_v1.wait()
        cast_group3(stage3_b, vg, 1)
        cp_v3 = load_qtr(v_ref, 3, stage3_b, sem_d)
        cp_wo.wait()
        wob[:, :] = stage_b[:, :].astype(jnp.bfloat16)
        cp_v2.wait()
        cast_group3(stage3_a, vg, 2)
        cp_v3.wait()
        cast_group3(stage3_b, vg, 3)

        def attn_part(c, qoff, nq):
            qrows = pl.ds(c * GRP + qoff, nq)
            kvrows = pl.ds(c * GRP, GRP)
            for h in range(HQ):
                cols = slice(h * DH, (h + 1) * DH)
                q = qg[qrows, cols]
                k = kg[kvrows, cols]
                s = lax.dot_general(q, k, (((1,), (1,)), ((), ())),
                                    preferred_element_type=jnp.float32)
                w = jnp.exp(s)
                acc[qrows, DM + h] = jnp.sum(w, axis=1)
                acc[qrows, cols] = lax.dot_general(
                    w.astype(jnp.bfloat16), vg[kvrows, cols],
                    (((1,), (0,)), ((), ())),
                    preferred_element_type=jnp.float32)

        def attn_chunk(c):
            attn_part(c, 0, GRP)

        attn_part(my, 0, HGRP)
        attn_part(lax.rem(my + 2, N_DEV), HGRP, HGRP)

        def rs_pair(slot):
            r_ = pltpu.make_async_remote_copy(
                src_ref=sbr.at[slot], dst_ref=rbr.at[slot],
                send_sem=send_sems.at[slot], recv_sem=recv_sems.at[slot],
                device_id=(right,), device_id_type=pl.DeviceIdType.MESH)
            l_ = pltpu.make_async_remote_copy(
                src_ref=sbl.at[slot], dst_ref=rbl.at[slot],
                send_sem=send_sems.at[2 + slot], recv_sem=recv_sems.at[2 + slot],
                device_id=(left,), device_id_type=pl.DeviceIdType.MESH)
            return r_, l_

        sbr[0, :, :] = acc[pl.ds(my * GRP, HGRP), :].astype(jnp.bfloat16)
        sbl[0, :, :] = acc[pl.ds(lax.rem(my + 2, N_DEV) * GRP + HGRP, HGRP),
                           :].astype(jnp.bfloat16)
        rdma_r0, rdma_l0 = rs_pair(0)
        rdma_r0.start()
        rdma_l0.start()
        attn_part(my, HGRP, HGRP)
        attn_part(lax.rem(my + 2, N_DEV), 0, HGRP)
        attn_chunk(lax.rem(my + 3, N_DEV))
        rdma_r0.wait()
        rdma_l0.wait()

        c1 = lax.rem(my + 3, N_DEV)
        sbr[1, :, :] = (rbr[0, :, :].astype(jnp.float32)
                        + acc[pl.ds(c1 * GRP, HGRP), :]).astype(jnp.bfloat16)
        sbl[1, :, :] = (rbl[0, :, :].astype(jnp.float32)
                        + acc[pl.ds(c1 * GRP + HGRP, HGRP), :]
                        ).astype(jnp.bfloat16)
        rdma_r1, rdma_l1 = rs_pair(1)
        rdma_r1.start()
        rdma_l1.start()
        attn_chunk(lax.rem(my + 1, N_DEV))
        rdma_r1.wait()
        rdma_l1.wait()

        sbr[0, :, :] = (rbr[1, :, :].astype(jnp.float32)
                        + acc[pl.ds(lax.rem(my + 2, N_DEV) * GRP, HGRP), :]
                        ).astype(jnp.bfloat16)
        sbl[0, :, :] = (rbl[1, :, :].astype(jnp.float32)
                        + acc[pl.ds(my * GRP + HGRP, HGRP), :]
                        ).astype(jnp.bfloat16)
        rdma_r2, rdma_l2 = rs_pair(0)
        rdma_r2.start()
        rdma_l2.start()
        rdma_r2.wait()
        rdma_l2.wait()

        own = lax.rem(my + 1, N_DEV)

        def norm_proj(rb, row_off):
            s_half = (rb[0, :, :].astype(jnp.float32)
                      + acc[pl.ds(own * GRP + row_off, HGRP), :])
            ctx = jnp.concatenate(
                [(s_half[:, h * DH:(h + 1) * DH]
                  * (1.0 / s_half[:, DM + h])[:, None]).astype(jnp.bfloat16)
                 for h in range(HQ)], axis=1)
            return lax.dot_general(
                ctx, wob[:, :], (((1,), (0,)), ((), ())),
                preferred_element_type=jnp.float32)

        def ag_pair(ss, rr):
            r_ = pltpu.make_async_remote_copy(
                src_ref=agr.at[ss], dst_ref=agr.at[rr],
                send_sem=send_sems.at[4 + ss], recv_sem=recv_sems.at[4 + rr],
                device_id=(right,), device_id_type=pl.DeviceIdType.MESH)
            l_ = pltpu.make_async_remote_copy(
                src_ref=agl.at[ss], dst_ref=agl.at[rr],
                send_sem=send_sems.at[6 + ss], recv_sem=recv_sems.at[6 + rr],
                device_id=(left,), device_id_type=pl.DeviceIdType.MESH)
            return r_, l_

        def ag_store(slot, t):
            rho_r = lax.rem(my - t + 2 * N_DEV, N_DEV)
            rho_l = lax.rem(my + t + 2, N_DEV)
            for j in range(NJ // 2):
                out_ref[0, pl.ds(j * NRES * BLK + rho_r * BLK, BLK), :] = \
                    agr[slot, pl.ds(j * BLK, BLK), :].astype(jnp.float32)
                out_ref[0, pl.ds((j + 4) * NRES * BLK + rho_l * BLK, BLK), :] = \
                    agl[slot, pl.ds(j * BLK, BLK), :].astype(jnp.float32)

        out_top = norm_proj(rbr, 0)
        agr[0, :, :] = out_top.astype(jnp.bfloat16)
        ag_r0, ag_l0 = ag_pair(0, 1)
        ag_r0.start()
        out_bot = norm_proj(rbl, HGRP)
        agl[0, :, :] = out_bot.astype(jnp.bfloat16)
        ag_l0.start()
        for j in range(NJ // 2):
            out_ref[0, pl.ds(j * NRES * BLK + own * BLK, BLK), :] = \
                out_top[j * BLK:(j + 1) * BLK, :]
            out_ref[0, pl.ds((j + 4) * NRES * BLK + own * BLK, BLK), :] = \
                out_bot[j * BLK:(j + 1) * BLK, :]
        ag_r0.wait()
        ag_l0.wait()
        ag_r1, ag_l1 = ag_pair(1, 0)
        ag_r1.start()
        ag_l1.start()
        ag_store(1, 0)
        ag_r1.wait()
        ag_l1.wait()
        ag_r2, ag_l2 = ag_pair(0, 1)
        ag_r2.start()
        ag_l2.start()
        ag_store(0, 1)
        ag_r2.wait()
        ag_l2.wait()
        ag_store(1, 2)

    out = pl.pallas_call(
        body,
        out_shape=jax.ShapeDtypeStruct((1, SQ, DM), jnp.float32),
        in_specs=[pl.BlockSpec(memory_space=pltpu.MemorySpace.HBM)] * 5,
        out_specs=pl.BlockSpec(memory_space=pltpu.VMEM),
        scratch_shapes=[
            pltpu.VMEM((HALF, DM), jnp.float32),
            pltpu.VMEM((HALF, DM), jnp.float32),
            pltpu.VMEM((QTR, HQ, DH), jnp.float32),
            pltpu.VMEM((QTR, HQ, DH), jnp.float32),
            pltpu.VMEM((DM, DM), jnp.bfloat16),
            pltpu.VMEM((DM, DM), jnp.bfloat16),
            pltpu.VMEM((SQ, DM), jnp.bfloat16),
            pltpu.VMEM((SQ, DM), jnp.bfloat16),
            pltpu.VMEM((SKV, DM), jnp.bfloat16),
            pltpu.VMEM((SKV, DM), jnp.bfloat16),
            pltpu.VMEM((SQ, PW), jnp.float32),
            pltpu.VMEM((2, HGRP, PW), jnp.bfloat16),
            pltpu.VMEM((2, HGRP, PW), jnp.bfloat16),
            pltpu.VMEM((2, HGRP, PW), jnp.bfloat16),
            pltpu.VMEM((2, HGRP, PW), jnp.bfloat16),
            pltpu.VMEM((2, HGRP, DM), jnp.bfloat16),
            pltpu.VMEM((2, HGRP, DM), jnp.bfloat16),
            pltpu.SemaphoreType.DMA((4,)),
            pltpu.SemaphoreType.DMA((8,)),
            pltpu.SemaphoreType.DMA((8,)),
        ],
        compiler_params=pltpu.CompilerParams(
            collective_id=0, vmem_limit_bytes=60 * 1024 * 1024),
    )(x2, Wq, K2, V2, Wo)

    return out
